# baseline (speedup 1.0000x reference)
import functools

import jax
import jax.numpy as jnp
from jax import lax
from jax.experimental import pallas as pl
from jax.experimental.pallas import tpu as pltpu
from jax.experimental.pallas import tpu_sc as plsc

N_LAYERS = 12
BEAM = 3
TOPK = 3
VOCAB = 100000
HIST = 20
KV_CHUNKS = 8
NEG = -3.4e38


def _beam_body(logits_ref, save_id_ref, rp_ref, prev_ref, pen_ref,
               tbi_ref, nsi_ref, rp_out_ref, tbp_ref, mli_ref, srcrows_ref,
               cand_v, cand_i):
    x = logits_ref[...] * rp_ref[...]
    m = jnp.max(x, axis=1, keepdims=True)
    lse = jnp.log(jnp.sum(jnp.exp(x - m), axis=1, keepdims=True))
    lg = x - m - lse

    iota = lax.broadcasted_iota(jnp.int32, (BEAM, VOCAB), 1)
    cur = lg
    for k in range(TOPK):
        mx = jnp.max(cur, axis=1, keepdims=True)
        am = jnp.min(jnp.where(cur == mx, iota, VOCAB), axis=1,
                     keepdims=True)
        for r in range(BEAM):
            cand_v[r * TOPK + k] = mx[r, 0] + prev_ref[r, 0]
            cand_i[r * TOPK + k] = am[r, 0]
        if k < TOPK - 1:
            cur = jnp.where(iota == am, NEG, cur)

    col_iota = lax.broadcasted_iota(jnp.int32, (1, VOCAB), 1)
    b_sel = []
    for j in range(BEAM):
        bv = cand_v[0]
        bc = jnp.int32(0)
        for c in range(1, BEAM * TOPK):
            take = cand_v[c] > bv
            bv = jnp.where(take, cand_v[c], bv)
            bc = jnp.where(take, jnp.int32(c), bc)
        cand_v[bc] = NEG
        b_j = bc // TOPK
        t_j = cand_i[bc]
        b_sel.append(b_j)
        tbp_ref[j, 0] = bv
        tbi_ref[j, 0] = t_j
        if j == 0:
            mli_ref[0] = t_j
        for t in range(HIST):
            nsi_ref[j, t] = save_id_ref[b_j, t]
        nsi_ref[j, HIST] = t_j
        row = rp_ref[pl.ds(b_j, 1), :]
        row = jnp.where(col_iota == t_j, row * pen_ref[0], row)
        rp_out_ref[pl.ds(j, 1), :] = row

    g_io = lax.broadcasted_iota(jnp.int32, (96, 16), 0)
    l_io = lax.broadcasted_iota(jnp.int32, (96, 16), 1)
    r_mat = g_io * 16 + l_io
    beam_mat = r_mat // 512
    rem_mat = r_mat - beam_mat * 512
    src_mat = jnp.where(beam_mat == 0, b_sel[0],
                        jnp.where(beam_mat == 1, b_sel[1], b_sel[2]))
    srcrows_ref[...] = src_mat * 512 + rem_mat


def _gather_body(bidx_ref, *refs):
    del bidx_ref
    n = len(refs) // 2
    for l in range(n):
        refs[n + l][...] = refs[l][...]


def _gather_tc(kvs, beam_index):
    n = len(kvs)
    shape = kvs[0].shape
    flat = shape[1] * shape[2] * shape[3]
    rows = flat // 128
    chunk = rows // KV_CHUNKS
    kvs2 = [kv.reshape(BEAM, rows, 128) for kv in kvs]

    def in_map(b, c, bidx):
        return (bidx[b], c, 0)

    def out_map(b, c, bidx):
        return (b, c, 0)

    block = (1, chunk, 128)
    grid_spec = pltpu.PrefetchScalarGridSpec(
        num_scalar_prefetch=1,
        grid=(BEAM, KV_CHUNKS),
        in_specs=[pl.BlockSpec(block, in_map) for _ in range(n)],
        out_specs=[pl.BlockSpec(block, out_map) for _ in range(n)],
    )
    outs = pl.pallas_call(
        _gather_body,
        grid_spec=grid_spec,
        out_shape=[jax.ShapeDtypeStruct((BEAM, rows, 128), kv.dtype)
                   for kv in kvs2],
    )(beam_index, *kvs2)
    return [o.reshape(shape) for o in outs]


def _gather_sc(kvs, src_rows):
    n = len(kvs)
    shape = kvs[0].shape
    flat = shape[1] * shape[2] * shape[3]
    E = 1536
    rpb = flat // E
    rows = BEAM * rpb
    nw = 32
    rpt = rows // nw
    nc = rpt // 16
    nb = 4
    T = n * nc
    kvs2 = [kv.reshape(rows, E) for kv in kvs]
    mesh = plsc.VectorSubcoreMesh(core_axis_name="c", subcore_axis_name="s")

    @functools.partial(
        pl.kernel,
        out_type=[jax.ShapeDtypeStruct((rows, E), jnp.float32)
                  for _ in range(n)],
        mesh=mesh,
        scratch_types=[
            [pltpu.VMEM((16,), jnp.int32) for _ in range(nc)],
            pltpu.VMEM((nb * 16, E), jnp.float32),
            pltpu.SemaphoreType.DMA((nb,)),
            pltpu.SemaphoreType.DMA((nb,)),
        ])
    def k(sr_hbm, *refs):
        kv_refs = refs[:n]
        out_refs = refs[n:2 * n]
        idx_scr, ring, in_sems, out_sems = refs[2 * n:]
        wid = lax.axis_index("s") * 2 + lax.axis_index("c")
        for c in range(nc):
            pltpu.sync_copy(sr_hbm.at[wid * nc + c], idx_scr[c])

        gh = [None] * T
        oh = [None] * T

        def start_out(t):
            l, c = divmod(t, nc)
            row = wid * rpt + c * 16
            oh[t] = pltpu.async_copy(
                ring.at[pl.ds((t % nb) * 16, 16)],
                out_refs[l].at[pl.ds(row, 16)],
                out_sems.at[t % nb])

        for t in range(T):
            l, c = divmod(t, nc)
            if t >= nb:
                oh[t - nb].wait()
            gh[t] = pltpu.async_copy(
                kv_refs[l].at[idx_scr[c]],
                ring.at[pl.ds((t % nb) * 16, 16)],
                in_sems.at[t % nb])
            if t >= 1:
                gh[t - 1].wait()
                start_out(t - 1)
        gh[T - 1].wait()
        start_out(T - 1)
        for t in range(T - nb, T):
            oh[t].wait()

    outs = k(src_rows, *kvs2)
    return [o.reshape(shape) for o in outs]


@jax.jit
def _run(kvs, logits, save_id, repeat_penality, previous_prob, penality_value):
    small_out_shape = [
        jax.ShapeDtypeStruct((BEAM, 1), jnp.int32),
        jax.ShapeDtypeStruct((BEAM, HIST + 1), jnp.int32),
        jax.ShapeDtypeStruct((BEAM, VOCAB), jnp.float32),
        jax.ShapeDtypeStruct((BEAM, 1), jnp.float32),
        jax.ShapeDtypeStruct((1,), jnp.int32),
        jax.ShapeDtypeStruct((96, 16), jnp.int32),
    ]
    vmem = pl.BlockSpec(memory_space=pltpu.MemorySpace.VMEM)
    smem = pl.BlockSpec(memory_space=pltpu.SMEM)
    tbi, nsi, rp_out, tbp, mli, src_rows = pl.pallas_call(
        _beam_body,
        out_shape=small_out_shape,
        in_specs=[vmem, smem, vmem, smem, smem],
        out_specs=[smem, smem, vmem, smem, smem, vmem],
        scratch_shapes=[
            pltpu.SMEM((BEAM * TOPK,), jnp.float32),
            pltpu.SMEM((BEAM * TOPK,), jnp.int32),
        ],
    )(logits, save_id, repeat_penality, previous_prob, penality_value)
    save_kv = _gather_sc(kvs, src_rows)
    return (*save_kv, tbi, nsi, rp_out, tbp, mli)


def kernel(kv_0, kv_1, kv_2, kv_3, kv_4, kv_5, kv_6, kv_7, kv_8, kv_9,
           kv_10, kv_11, logits, save_id, repeat_penality, previous_prob,
           penality_value, beam_size, topK):
    kvs = (kv_0, kv_1, kv_2, kv_3, kv_4, kv_5, kv_6, kv_7, kv_8, kv_9,
           kv_10, kv_11)
    return _run(kvs, logits, save_id, repeat_penality, previous_prob,
                penality_value)


# speedup vs baseline: 1.0879x; 1.0879x over previous
import functools

import jax
import jax.numpy as jnp
from jax import lax
from jax.experimental import pallas as pl
from jax.experimental.pallas import tpu as pltpu
from jax.experimental.pallas import tpu_sc as plsc

N_LAYERS = 12
BEAM = 3
TOPK = 3
VOCAB = 100000
HIST = 20
KV_CHUNKS = 8
NEG = -3.4e38


def _beam_body(logits_ref, save_id_ref, rp_ref, prev_ref, pen_ref,
               tbi_ref, nsi_ref, rp_out_ref, tbp_ref, mli_ref, srcrows_ref,
               cand_v, cand_i):
    x = logits_ref[...] * rp_ref[...]
    m = jnp.max(x, axis=1, keepdims=True)
    lse = jnp.log(jnp.sum(jnp.exp(x - m), axis=1, keepdims=True))
    lg = x - m - lse

    iota = lax.broadcasted_iota(jnp.int32, (BEAM, VOCAB), 1)
    cur = lg
    for k in range(TOPK):
        mx = jnp.max(cur, axis=1, keepdims=True)
        am = jnp.min(jnp.where(cur == mx, iota, VOCAB), axis=1,
                     keepdims=True)
        for r in range(BEAM):
            cand_v[r * TOPK + k] = mx[r, 0] + prev_ref[r, 0]
            cand_i[r * TOPK + k] = am[r, 0]
        if k < TOPK - 1:
            cur = jnp.where(iota == am, NEG, cur)

    col_iota = lax.broadcasted_iota(jnp.int32, (1, VOCAB), 1)
    b_sel = []
    for j in range(BEAM):
        bv = cand_v[0]
        bc = jnp.int32(0)
        for c in range(1, BEAM * TOPK):
            take = cand_v[c] > bv
            bv = jnp.where(take, cand_v[c], bv)
            bc = jnp.where(take, jnp.int32(c), bc)
        cand_v[bc] = NEG
        b_j = bc // TOPK
        t_j = cand_i[bc]
        b_sel.append(b_j)
        tbp_ref[j, 0] = bv
        tbi_ref[j, 0] = t_j
        if j == 0:
            mli_ref[0] = t_j
        for t in range(HIST):
            nsi_ref[j, t] = save_id_ref[b_j, t]
        nsi_ref[j, HIST] = t_j
        row = rp_ref[pl.ds(b_j, 1), :]
        row = jnp.where(col_iota == t_j, row * pen_ref[0], row)
        rp_out_ref[pl.ds(j, 1), :] = row

    packed = b_sel[0] + 4 * b_sel[1] + 16 * b_sel[2]
    for j in range(16):
        srcrows_ref[j] = packed


def _gather_body(bidx_ref, *refs):
    del bidx_ref
    n = len(refs) // 2
    for l in range(n):
        refs[n + l][...] = refs[l][...]


def _gather_tc(kvs, beam_index):
    n = len(kvs)
    shape = kvs[0].shape
    flat = shape[1] * shape[2] * shape[3]
    rows = flat // 128
    chunk = rows // KV_CHUNKS
    kvs2 = [kv.reshape(BEAM, rows, 128) for kv in kvs]

    def in_map(b, c, bidx):
        return (bidx[b], c, 0)

    def out_map(b, c, bidx):
        return (b, c, 0)

    block = (1, chunk, 128)
    grid_spec = pltpu.PrefetchScalarGridSpec(
        num_scalar_prefetch=1,
        grid=(BEAM, KV_CHUNKS),
        in_specs=[pl.BlockSpec(block, in_map) for _ in range(n)],
        out_specs=[pl.BlockSpec(block, out_map) for _ in range(n)],
    )
    outs = pl.pallas_call(
        _gather_body,
        grid_spec=grid_spec,
        out_shape=[jax.ShapeDtypeStruct((BEAM, rows, 128), kv.dtype)
                   for kv in kvs2],
    )(beam_index, *kvs2)
    return [o.reshape(shape) for o in outs]


def _gather_sc(kvs, beam_index16):
    n = len(kvs)
    shape = kvs[0].shape
    nh, sl, hd = shape[1], shape[2], shape[3]
    ch = 128
    nchunk = sl // ch
    items = BEAM * nh * nchunk
    nw = 32
    ipt = items // nw
    nb = 7
    T = n * ipt
    mesh = plsc.VectorSubcoreMesh(core_axis_name="c", subcore_axis_name="s")

    @functools.partial(
        pl.kernel,
        out_type=[jax.ShapeDtypeStruct(shape, jnp.float32)
                  for _ in range(n)],
        mesh=mesh,
        compiler_params=pltpu.CompilerParams(needs_layout_passes=False),
        scratch_types=[
            pltpu.VMEM((16,), jnp.int32),
            pltpu.VMEM((nb * ch, hd), jnp.float32),
            pltpu.SemaphoreType.DMA((nb,)),
            pltpu.SemaphoreType.DMA((nb,)),
        ])
    def k(bi_hbm, *refs):
        kv_refs = refs[:n]
        out_refs = refs[n:2 * n]
        bi_v, ring, in_sems, out_sems = refs[2 * n:]
        wid = lax.axis_index("s") * 2 + lax.axis_index("c")
        pltpu.sync_copy(bi_hbm, bi_v)
        packed = jnp.max(bi_v[...])
        b_sc = [packed & 3, (packed >> 2) & 3, (packed >> 4) & 3]

        coords = []
        for i in range(ipt):
            q = wid * ipt + i
            pair = q // nchunk
            cc = q - pair * nchunk
            j = pair // nh
            h = pair - j * nh
            b_src = jnp.where(j == 0, b_sc[0],
                              jnp.where(j == 1, b_sc[1], b_sc[2]))
            coords.append((j, h, cc, b_src))

        gh = [None] * T
        oh = [None] * T

        def start_out(t):
            l, i = divmod(t, ipt)
            j, h, cc, _ = coords[i]
            oh[t] = pltpu.async_copy(
                ring.at[pl.ds((t % nb) * ch, ch)],
                out_refs[l].at[j, h, pl.ds(cc * ch, ch), :],
                out_sems.at[t % nb])

        for t in range(T):
            l, i = divmod(t, ipt)
            j, h, cc, b_src = coords[i]
            if t >= nb:
                oh[t - nb].wait()
            gh[t] = pltpu.async_copy(
                kv_refs[l].at[b_src, h, pl.ds(cc * ch, ch), :],
                ring.at[pl.ds((t % nb) * ch, ch)],
                in_sems.at[t % nb])
            if t >= 1:
                gh[t - 1].wait()
                start_out(t - 1)
        gh[T - 1].wait()
        start_out(T - 1)
        for t in range(T - nb, T):
            oh[t].wait()

    return list(k(beam_index16, *kvs))


@jax.jit
def _run(kvs, logits, save_id, repeat_penality, previous_prob, penality_value):
    small_out_shape = [
        jax.ShapeDtypeStruct((BEAM, 1), jnp.int32),
        jax.ShapeDtypeStruct((BEAM, HIST + 1), jnp.int32),
        jax.ShapeDtypeStruct((BEAM, VOCAB), jnp.float32),
        jax.ShapeDtypeStruct((BEAM, 1), jnp.float32),
        jax.ShapeDtypeStruct((1,), jnp.int32),
        jax.ShapeDtypeStruct((16,), jnp.int32),
    ]
    vmem = pl.BlockSpec(memory_space=pltpu.MemorySpace.VMEM)
    smem = pl.BlockSpec(memory_space=pltpu.SMEM)
    tbi, nsi, rp_out, tbp, mli, src_rows = pl.pallas_call(
        _beam_body,
        out_shape=small_out_shape,
        in_specs=[vmem, smem, vmem, smem, smem],
        out_specs=[smem, smem, vmem, smem, smem, smem],
        scratch_shapes=[
            pltpu.SMEM((BEAM * TOPK,), jnp.float32),
            pltpu.SMEM((BEAM * TOPK,), jnp.int32),
        ],
    )(logits, save_id, repeat_penality, previous_prob, penality_value)
    save_kv = _gather_sc(kvs, src_rows)
    return (*save_kv, tbi, nsi, rp_out, tbp, mli)


def kernel(kv_0, kv_1, kv_2, kv_3, kv_4, kv_5, kv_6, kv_7, kv_8, kv_9,
           kv_10, kv_11, logits, save_id, repeat_penality, previous_prob,
           penality_value, beam_size, topK):
    kvs = (kv_0, kv_1, kv_2, kv_3, kv_4, kv_5, kv_6, kv_7, kv_8, kv_9,
           kv_10, kv_11)
    return _run(kvs, logits, save_id, repeat_penality, previous_prob,
                penality_value)
